# baseline (speedup 1.0000x reference)
import functools

import jax
import jax.numpy as jnp
from jax import lax
from jax.experimental import pallas as pl
from jax.experimental.pallas import tpu as pltpu
from jax.experimental.pallas import tpu_sc as plsc

D = 64
DP = 128
W = 128
NW = 32
MAX_NORM = 1.0


def _rsqrt_nr(x):
    i = lax.bitcast_convert_type(x, jnp.int32)
    i = jnp.int32(0x5F3759DF) - lax.shift_right_logical(i, 1)
    y = lax.bitcast_convert_type(i, jnp.float32)
    for _ in range(3):
        y = y * (jnp.float32(1.5) - jnp.float32(0.5) * x * y * y)
    return y


def kernel(xc_padded, table):
    B, S = xc_padded.shape
    n = B * S
    per_w = n // NW
    nwin = per_w // W
    idx = xc_padded.reshape(n // DP, DP)
    table_p = jnp.pad(table, ((0, 0), (0, DP - D)))
    mesh = plsc.VectorSubcoreMesh(core_axis_name="core", subcore_axis_name="subcore")
    cp = pltpu.CompilerParams(
        needs_layout_passes=False, use_tc_tiling_on_sc=True
    )

    @functools.partial(
        pl.kernel,
        out_type=jax.ShapeDtypeStruct((n * D // DP, DP), jnp.float32),
        mesh=mesh,
        compiler_params=cp,
        scratch_types=[
            pltpu.VMEM((per_w // DP, DP), jnp.int32),
            pltpu.VMEM((2, W, DP), jnp.float32),
            pltpu.VMEM((2, W * D // DP, DP), jnp.float32),
            pltpu.SemaphoreType.DMA,
            pltpu.SemaphoreType.DMA,
            pltpu.SemaphoreType.DMA,
        ],
    )
    def k(table_hbm, idx_hbm, out_hbm, idx_v, buf, obuf, isem, gsem, osem):
        wid = lax.axis_index("subcore") * 2 + lax.axis_index("core")
        pltpu.async_copy(
            idx_hbm.at[pl.ds(wid * (per_w // DP), per_w // DP)], idx_v, isem
        ).wait()
        lanes = lax.iota(jnp.int32, 16)

        def fire_gathers(win, slot):
            for j in range(W // 16):
                iv = idx_v[win, pl.ds(j * 16, 16)]
                pltpu.async_copy(
                    table_hbm.at[iv], buf.at[slot, pl.ds(j * 16, 16)], gsem
                )

        def drain_gathers(slot):
            pltpu.make_async_copy(
                table_hbm.at[pl.ds(0, W)], buf.at[slot], gsem
            ).wait()

        def compute(slot):
            @pl.loop(0, W // 16)
            def _(g):
                rows = lanes + g * 16
                s0 = jnp.zeros((16,), jnp.float32)
                s1 = jnp.zeros((16,), jnp.float32)
                s2 = jnp.zeros((16,), jnp.float32)
                s3 = jnp.zeros((16,), jnp.float32)
                accs = [s0, s1, s2, s3]
                for c in range(D):
                    cols = jnp.full((16,), c, jnp.int32)
                    v = plsc.load_gather(buf.at[slot], [rows, cols])
                    accs[c % 4] = accs[c % 4] + v * v
                sumsq = (accs[0] + accs[1]) + (accs[2] + accs[3])
                scale16 = jnp.where(
                    sumsq > jnp.float32(MAX_NORM * MAX_NORM),
                    jnp.float32(MAX_NORM) * _rsqrt_nr(sumsq),
                    jnp.float32(1.0),
                )
                for r in range(16):
                    src = buf.at[slot, g * 16 + r]
                    sc = scale16[r]
                    orow = g * 8 + r // 2
                    ocol = (r % 2) * D
                    for c4 in range(4):
                        obuf[slot, orow, pl.ds(ocol + c4 * 16, 16)] = (
                            src[pl.ds(c4 * 16, 16)] * sc
                        )

        fire_gathers(0, 0)

        @pl.loop(0, nwin)
        def _(w):
            slot = lax.rem(w, 2)
            nslot = 1 - slot
            with jax.named_scope("drain_gathers"):
                drain_gathers(slot)

            @pl.when(w + 1 < nwin)
            def _():
                with jax.named_scope("fire_gathers"):
                    fire_gathers(w + 1, nslot)

            @pl.when(w >= 2)
            def _():
                with jax.named_scope("wait_write"):
                    pltpu.make_async_copy(
                        obuf.at[slot], out_hbm.at[pl.ds(0, W * D // DP)], osem
                    ).wait()

            with jax.named_scope("compute"):
                compute(slot)
            rowbase = (wid * nwin + w) * (W * D // DP)
            pltpu.async_copy(
                obuf.at[slot], out_hbm.at[pl.ds(rowbase, W * D // DP)], osem
            )

        pltpu.make_async_copy(
            obuf.at[0], out_hbm.at[pl.ds(0, W * D // DP)], osem
        ).wait()
        pltpu.make_async_copy(
            obuf.at[1], out_hbm.at[pl.ds(0, W * D // DP)], osem
        ).wait()

    out = k(table_p, idx)
    return out.reshape(B, S, D)


# speedup vs baseline: 1.1206x; 1.1206x over previous
import functools

import jax
import jax.numpy as jnp
from jax import lax
from jax.experimental import pallas as pl
from jax.experimental.pallas import tpu as pltpu
from jax.experimental.pallas import tpu_sc as plsc

D = 64
DP = 128
W = 128
NW = 32
MAX_NORM = 1.0


def _rsqrt_nr(x):
    i = lax.bitcast_convert_type(x, jnp.int32)
    i = jnp.int32(0x5F3759DF) - lax.shift_right_logical(i, 1)
    y = lax.bitcast_convert_type(i, jnp.float32)
    for _ in range(3):
        y = y * (jnp.float32(1.5) - jnp.float32(0.5) * x * y * y)
    return y


def kernel(xc_padded, table):
    B, S = xc_padded.shape
    n = B * S
    per_w = n // NW
    nwin = per_w // W
    idx = xc_padded.reshape(n // DP, DP)
    table_p = jnp.pad(table, ((0, 0), (0, DP - D)))
    mesh = plsc.VectorSubcoreMesh(core_axis_name="core", subcore_axis_name="subcore")
    cp = pltpu.CompilerParams(
        needs_layout_passes=False, use_tc_tiling_on_sc=True
    )

    @functools.partial(
        pl.kernel,
        out_type=jax.ShapeDtypeStruct((n * D // DP, DP), jnp.float32),
        mesh=mesh,
        compiler_params=cp,
        scratch_types=[
            pltpu.VMEM((per_w // DP, DP), jnp.int32),
            pltpu.VMEM((2, W, DP), jnp.float32),
            pltpu.VMEM((2, W * D // DP, DP), jnp.float32),
            pltpu.VMEM((16, 16), jnp.float32),
            pltpu.SemaphoreType.DMA,
            pltpu.SemaphoreType.DMA,
            pltpu.SemaphoreType.DMA,
        ],
    )
    def k(table_hbm, idx_hbm, out_hbm, idx_v, buf, obuf, tscr, isem, gsem, osem):
        wid = lax.axis_index("subcore") * 2 + lax.axis_index("core")
        pltpu.async_copy(
            idx_hbm.at[pl.ds(wid * (per_w // DP), per_w // DP)], idx_v, isem
        ).wait()
        lanes = lax.iota(jnp.int32, 16)

        def fire_gathers(win, slot):
            for j in range(W // 16):
                iv = idx_v[win, pl.ds(j * 16, 16)]
                pltpu.async_copy(
                    table_hbm.at[iv], buf.at[slot, pl.ds(j * 16, 16)], gsem
                )

        def drain_gathers(slot):
            pltpu.make_async_copy(
                table_hbm.at[pl.ds(0, W)], buf.at[slot], gsem
            ).wait()

        def compute(slot):
            @pl.loop(0, W // 16)
            def _(g):
                for r in range(16):
                    src = buf.at[slot, g * 16 + r]
                    v0 = src[pl.ds(0, 16)]
                    v1 = src[pl.ds(16, 16)]
                    v2 = src[pl.ds(32, 16)]
                    v3 = src[pl.ds(48, 16)]
                    s = (v0 * v0 + v1 * v1) + (v2 * v2 + v3 * v3)
                    cols = jnp.full((16,), r, jnp.int32)
                    plsc.store_scatter(tscr, [lanes, cols], s)
                sumsq = tscr[0]
                for j in range(1, 16):
                    sumsq = sumsq + tscr[j]
                scale16 = jnp.where(
                    sumsq > jnp.float32(MAX_NORM * MAX_NORM),
                    jnp.float32(MAX_NORM) * _rsqrt_nr(sumsq),
                    jnp.float32(1.0),
                )
                for r in range(16):
                    src = buf.at[slot, g * 16 + r]
                    sc = scale16[r]
                    orow = g * 8 + r // 2
                    ocol = (r % 2) * D
                    for c4 in range(4):
                        obuf[slot, orow, pl.ds(ocol + c4 * 16, 16)] = (
                            src[pl.ds(c4 * 16, 16)] * sc
                        )

        fire_gathers(0, 0)

        @pl.loop(0, nwin)
        def _(w):
            slot = lax.rem(w, 2)
            nslot = 1 - slot
            with jax.named_scope("drain_gathers"):
                drain_gathers(slot)

            @pl.when(w + 1 < nwin)
            def _():
                with jax.named_scope("fire_gathers"):
                    fire_gathers(w + 1, nslot)

            @pl.when(w >= 2)
            def _():
                with jax.named_scope("wait_write"):
                    pltpu.make_async_copy(
                        obuf.at[slot], out_hbm.at[pl.ds(0, W * D // DP)], osem
                    ).wait()

            with jax.named_scope("compute"):
                compute(slot)
            rowbase = (wid * nwin + w) * (W * D // DP)
            pltpu.async_copy(
                obuf.at[slot], out_hbm.at[pl.ds(rowbase, W * D // DP)], osem
            )

        pltpu.make_async_copy(
            obuf.at[0], out_hbm.at[pl.ds(0, W * D // DP)], osem
        ).wait()
        pltpu.make_async_copy(
            obuf.at[1], out_hbm.at[pl.ds(0, W * D // DP)], osem
        ).wait()

    out = k(table_p, idx)
    return out.reshape(B, S, D)


# speedup vs baseline: 1.2160x; 1.0851x over previous
import functools

import jax
import jax.numpy as jnp
from jax import lax
from jax.experimental import pallas as pl
from jax.experimental.pallas import tpu as pltpu
from jax.experimental.pallas import tpu_sc as plsc

D = 64
W = 128
NW = 32
NG = W // 16
MAX_NORM = 1.0


def _rsqrt_nr(x):
    i = lax.bitcast_convert_type(x, jnp.int32)
    i = jnp.int32(0x5F3759DF) - lax.shift_right_logical(i, 1)
    y = lax.bitcast_convert_type(i, jnp.float32)
    for _ in range(3):
        y = y * (jnp.float32(1.5) - jnp.float32(0.5) * x * y * y)
    return y


def kernel(xc_padded, table):
    B, S = xc_padded.shape
    n = B * S
    per_w = n // NW
    nwin = per_w // W
    idx = xc_padded.reshape(n // 128, 128)
    mesh = plsc.VectorSubcoreMesh(core_axis_name="core", subcore_axis_name="subcore")
    cp = pltpu.CompilerParams(
        needs_layout_passes=False, use_tc_tiling_on_sc=False
    )

    @functools.partial(
        pl.kernel,
        out_type=jax.ShapeDtypeStruct((n, D), jnp.float32),
        mesh=mesh,
        compiler_params=cp,
        scratch_types=[
            pltpu.VMEM((per_w // 128, 128), jnp.int32),
            pltpu.VMEM((2, W, D), jnp.float32),
            pltpu.VMEM((NG, 16, 16), jnp.float32),
            pltpu.SemaphoreType.DMA,
            pltpu.SemaphoreType.DMA,
            pltpu.SemaphoreType.DMA,
        ],
    )
    def k(table_hbm, idx_hbm, out_hbm, idx_v, buf, tscr, isem, gsem, osem):
        wid = lax.axis_index("subcore") * 2 + lax.axis_index("core")
        pltpu.async_copy(
            idx_hbm.at[pl.ds(wid * (per_w // 128), per_w // 128)], idx_v, isem
        ).wait()
        lanes = lax.iota(jnp.int32, 16)

        def fire_gathers(win, slot):
            for j in range(W // 16):
                iv = idx_v[win, pl.ds(j * 16, 16)]
                pltpu.async_copy(
                    table_hbm.at[iv], buf.at[slot, pl.ds(j * 16, 16)], gsem
                )

        def drain_gathers(slot):
            pltpu.make_async_copy(
                table_hbm.at[pl.ds(0, W)], buf.at[slot], gsem
            ).wait()

        def compute(slot):
            for g in range(NG):
                for r in range(16):
                    src = buf.at[slot, g * 16 + r]
                    v0 = src[pl.ds(0, 16)]
                    v1 = src[pl.ds(16, 16)]
                    v2 = src[pl.ds(32, 16)]
                    v3 = src[pl.ds(48, 16)]
                    s = (v0 * v0 + v1 * v1) + (v2 * v2 + v3 * v3)
                    cols = jnp.full((16,), r, jnp.int32)
                    plsc.store_scatter(tscr.at[g], [lanes, cols], s)
                sumsq = tscr[g, 0]
                for j in range(1, 16):
                    sumsq = sumsq + tscr[g, j]
                scale16 = jnp.where(
                    sumsq > jnp.float32(MAX_NORM * MAX_NORM),
                    jnp.float32(MAX_NORM) * _rsqrt_nr(sumsq),
                    jnp.float32(1.0),
                )
                for r in range(16):
                    row = buf.at[slot, g * 16 + r]
                    sc = scale16[r]
                    for c4 in range(4):
                        sl = pl.ds(c4 * 16, 16)
                        row[sl] = row[sl] * sc

        def step(w, slot):
            with jax.named_scope("drain_gathers"):
                drain_gathers(slot)

            @pl.when(w + 1 < nwin)
            def _():
                @pl.when(w >= 1)
                def _():
                    with jax.named_scope("wait_write"):
                        pltpu.make_async_copy(
                            buf.at[1 - slot], out_hbm.at[pl.ds(0, W)], osem
                        ).wait()

                with jax.named_scope("fire_gathers"):
                    fire_gathers(w + 1, 1 - slot)

            with jax.named_scope("compute"):
                compute(slot)
            rowbase = (wid * nwin + w) * W
            pltpu.async_copy(buf.at[slot], out_hbm.at[pl.ds(rowbase, W)], osem)

        fire_gathers(0, 0)

        @pl.loop(0, nwin // 2)
        def _(h):
            step(2 * h, 0)
            step(2 * h + 1, 1)

        pltpu.make_async_copy(buf.at[0], out_hbm.at[pl.ds(0, W)], osem).wait()
        pltpu.make_async_copy(buf.at[1], out_hbm.at[pl.ds(0, W)], osem).wait()

    out = k(table, idx)
    return out.reshape(B, S, D)
